# baseline (device time: 75623 ns/iter reference)
import jax
import jax.numpy as jnp
from jax import lax
from jax.experimental import pallas as pl
from jax.experimental.pallas import tpu as pltpu

N_DEV = 32
M = 1536
N = 1536
N_STREAMS = 3
COLS = N // N_STREAMS

ALL_MASKS = (1, 3, 4, 8, 16, 20)
HALVES = (768, 384, 192)
OFFS = (0, 768, 1152)
QUAD_OFFS = (1344, 1392, 1440)
QUAD_MASKS = (4, 16, 20)

ORDERS = (
    (1, 3, 8),
    (8, 1, 3),
    (3, 8, 1),
)

SEMS_PER_STREAM = 12


def _silu_f32(v):
    return v * (1.0 / (1.0 + jnp.exp(-v)))


def kernel(A, B):
    def body(a_ref, b_ref, out_ref, z_ref,
             zb_a, rs_a, ag_a, zb_b, rs_b, ag_b, zb_c, rs_c, ag_c,
             send_sems, recv_sems, exit_sem):
        my = lax.axis_index("i")

        def phi(mask):
            if mask == 1:
                return (my ^ (my >> 1)) & 1
            shift = {3: 1, 4: 2, 8: 3, 16: 4}[mask]
            return (my >> shift) & 1

        barrier = pltpu.get_barrier_semaphore()
        for m in ALL_MASKS:
            pl.semaphore_signal(
                barrier, inc=1, device_id=(my ^ m,),
                device_id_type=pl.DeviceIdType.MESH,
            )
        pl.semaphore_wait(barrier, len(ALL_MASKS))

        bufs = [(zb_a, rs_a, ag_a), (zb_b, rs_b, ag_b), (zb_c, rs_c, ag_c)]
        streams = [
            dict(order=ORDERS[t], col=t * COLS, zb=bufs[t][0],
                 rs=bufs[t][1], ag=bufs[t][2],
                 sem_base=SEMS_PER_STREAM * t, lo=0)
            for t in range(N_STREAMS)
        ]

        def rdma(st, mask, sem_idx, src, dst):
            k = st["sem_base"] + sem_idx
            return pltpu.make_async_remote_copy(
                src_ref=src,
                dst_ref=dst,
                send_sem=send_sems.at[k],
                recv_sem=recv_sems.at[k],
                device_id=(my ^ mask,),
                device_id_type=pl.DeviceIdType.MESH,
            )

        def matmul_piece(row_lo, col_lo):
            z_ref[pl.ds(row_lo, HALVES[0]), pl.ds(col_lo, COLS)] = jnp.dot(
                a_ref[pl.ds(row_lo, HALVES[0]), :].astype(jnp.bfloat16),
                b_ref[:, pl.ds(col_lo, COLS)].astype(jnp.bfloat16),
                preferred_element_type=jnp.float32,
            )

        def start_rs(st, stage, send_lo, half):
            st["zb"][pl.ds(send_lo, half), :] = (
                z_ref[pl.ds(send_lo, half),
                      pl.ds(st["col"], COLS)].astype(jnp.bfloat16)
            )
            op = rdma(
                st, st["order"][stage], stage,
                st["zb"].at[pl.ds(send_lo, half), :],
                st["rs"].at[pl.ds(OFFS[stage], half), :],
            )
            op.start()
            return op

        def add_recv(st, z_lo, stage_off, n_rows):
            z_ref[pl.ds(z_lo, n_rows), pl.ds(st["col"], COLS)] = (
                z_ref[pl.ds(z_lo, n_rows), pl.ds(st["col"], COLS)]
                + st["rs"][pl.ds(stage_off, n_rows), :].astype(jnp.float32)
            )

        for st in streams:
            b = phi(st["order"][0])
            send_lo = (1 - b) * HALVES[0]
            st["keep0"] = b * HALVES[0]
            matmul_piece(send_lo, st["col"])
            st["op"] = start_rs(st, 0, send_lo, HALVES[0])
        for st in streams:
            matmul_piece(st["keep0"], st["col"])
            st["lo"] = st["keep0"]

        for s in range(1, 3):
            half = HALVES[s]
            for st in streams:
                st["op"].wait()
                b = phi(st["order"][s])
                send_sub = (1 - b) * half
                keep_sub = b * half
                send_lo = st["lo"] + send_sub
                add_recv(st, send_lo, OFFS[s - 1] + send_sub, half)
                st["op"] = start_rs(st, s, send_lo, half)
                st["pend"] = (st["lo"] + keep_sub, OFFS[s - 1] + keep_sub)
                st["lo"] = st["lo"] + keep_sub
            for st in streams:
                add_recv(st, st["pend"][0], st["pend"][1], half)

        for st in streams:
            st["op"].wait()
            b4 = phi(4)
            b16 = phi(16)
            subs = {
                4: (1 - b4) * 96 + b16 * 48,
                16: b4 * 96 + (1 - b16) * 48,
                20: (1 - b4) * 96 + (1 - b16) * 48,
            }
            st["quad_ops"] = []
            for qi, m in enumerate(QUAD_MASKS):
                send_lo = st["lo"] + subs[m]
                add_recv(st, send_lo, OFFS[2] + subs[m], 48)
                st["zb"][pl.ds(send_lo, 48), :] = (
                    z_ref[pl.ds(send_lo, 48),
                          pl.ds(st["col"], COLS)].astype(jnp.bfloat16)
                )
                op = rdma(
                    st, m, 3 + qi,
                    st["zb"].at[pl.ds(send_lo, 48), :],
                    st["rs"].at[pl.ds(QUAD_OFFS[qi], 48), :],
                )
                op.start()
                st["quad_ops"].append(op)
            st["keep_sub"] = b4 * 96 + b16 * 48
        for st in streams:
            add_recv(st, st["lo"] + st["keep_sub"], OFFS[2] + st["keep_sub"], 48)

        for st in streams:
            for op in st["quad_ops"]:
                op.wait()
            own_lo = st["lo"] + st["keep_sub"]
            acc = z_ref[pl.ds(own_lo, 48), pl.ds(st["col"], COLS)]
            for qi in range(3):
                acc = acc + st["rs"][pl.ds(QUAD_OFFS[qi], 48), :].astype(
                    jnp.float32
                )
            own = _silu_f32(acc)
            out_ref[pl.ds(own_lo, 48), pl.ds(st["col"], COLS)] = own
            st["ag"][pl.ds(own_lo, 48), :] = own.astype(jnp.bfloat16)
            st["own_lo"] = own_lo
            st["win_lo"] = st["lo"]

        for st in streams:
            st["quad_ops"] = []
            for qi, m in enumerate(QUAD_MASKS):
                op = rdma(
                    st, m, 6 + qi,
                    st["ag"].at[pl.ds(st["own_lo"], 48), :],
                    st["ag"].at[pl.ds(st["own_lo"], 48), :],
                )
                op.start()
                st["quad_ops"].append(op)
        for st in streams:
            for op in st["quad_ops"]:
                op.wait()
            st["lo"] = st["win_lo"]
            st["pend"] = None

        first_pend = [(st["win_lo"], 192) for st in streams]
        for k, s in enumerate((2, 1, 0)):
            half = HALVES[s]
            for i, st in enumerate(streams):
                if k > 0:
                    st["op"].wait()
                b = phi(st["order"][s])
                op = rdma(
                    st, st["order"][s], 9 + k,
                    st["ag"].at[pl.ds(st["lo"], half), :],
                    st["ag"].at[pl.ds(st["lo"], half), :],
                )
                op.start()
                plo, phalf = first_pend[i] if k == 0 else st["pend"]
                out_ref[pl.ds(plo, phalf), pl.ds(st["col"], COLS)] = (
                    st["ag"][pl.ds(plo, phalf), :].astype(jnp.float32)
                )
                st["op"] = op
                st["pend"] = (st["lo"] + (1 - 2 * b) * half, half)
                st["lo"] = st["lo"] - b * half
        for st in streams:
            st["op"].wait()
            plo, phalf = st["pend"]
            out_ref[pl.ds(plo, phalf), pl.ds(st["col"], COLS)] = (
                st["ag"][pl.ds(plo, phalf), :].astype(jnp.float32)
            )

        for m in ALL_MASKS:
            pl.semaphore_signal(
                exit_sem, inc=1, device_id=(my ^ m,),
                device_id_type=pl.DeviceIdType.MESH,
            )
        pl.semaphore_wait(exit_sem, len(ALL_MASKS))

    stream_scratch = []
    for _ in range(N_STREAMS):
        stream_scratch += [
            pltpu.VMEM((M, COLS), jnp.bfloat16),
            pltpu.VMEM((M, COLS), jnp.bfloat16),
            pltpu.VMEM((M, COLS), jnp.bfloat16),
        ]

    n_sems = SEMS_PER_STREAM * N_STREAMS
    return pl.pallas_call(
        body,
        out_shape=jax.ShapeDtypeStruct((M, N), jnp.float32),
        in_specs=[
            pl.BlockSpec(memory_space=pltpu.VMEM),
            pl.BlockSpec(memory_space=pltpu.VMEM),
        ],
        out_specs=pl.BlockSpec(memory_space=pltpu.VMEM),
        scratch_shapes=[
            pltpu.VMEM((M, N), jnp.float32),
            *stream_scratch,
            pltpu.SemaphoreType.DMA((n_sems,)),
            pltpu.SemaphoreType.DMA((n_sems,)),
            pltpu.SemaphoreType.REGULAR,
        ],
        compiler_params=pltpu.CompilerParams(collective_id=0),
    )(A, B)
